# Initial kernel scaffold; baseline (speedup 1.0000x reference)
import jax
import jax.numpy as jnp
from jax.experimental import pallas as pl


def kernel(inp, params):
    raise NotImplementedError("write your pallas kernel here")


# speedup vs baseline: 1.3571x; 1.3571x over previous
import numpy as np
import jax
import jax.numpy as jnp
from jax.experimental import pallas as pl

_B = 1024
_V = 1024
_C = 32
_CH = 16
_A = 10
_NA = 16
_PN = (1, 2, 3, 4)
_BETA = 0.25
_RESI = 0.5
_H = 4

_pallas_call = pl.pallas_call
_HI = jax.lax.Precision.HIGHEST


def _hmap_s1(h):
    s = np.zeros((3, h, h), np.float32)
    for k in range(3):
        for ho in range(h):
            hi = ho + k - 1
            if 0 <= hi < h:
                s[k, hi, ho] = 1.0
    return s


def _hmap_s2(h):
    s = np.zeros((3, h, h // 2), np.float32)
    for k in range(3):
        for ho in range(h // 2):
            hi = 2 * ho + k
            if 0 <= hi < h:
                s[k, hi, ho] = 1.0
    return s


def _dup(h, c):
    d = np.zeros((h * c, 2 * h * c), np.float32)
    for ho in range(2 * h):
        for cc in range(c):
            d[(ho // 2) * c + cc, ho * c + cc] = 1.0
    return d


def _band(smap, w_stack):
    wt = w_stack[:, :, :, :, 1]
    m = jnp.einsum('kio,adck->aicod', jnp.asarray(smap), wt)
    a, hi, ci, ho, co = m.shape
    return m.reshape(a, hi * ci, ho * co)


def _btile(b_stack, ho):
    a, co = b_stack.shape
    return jnp.broadcast_to(b_stack[:, None, :], (a, ho, co)).reshape(a, 1, ho * co)


def _bdot(x, m):
    return jnp.dot(x.astype(jnp.bfloat16), m, preferred_element_type=jnp.float32)


def _body(x_ref,
          m1, m2, m3, m4, m5, m6, mq, mpq, d1, d2, d3, d4, d5, d6,
          b1, b2, b3, b4, b5, b6, bq, bpq, c1, c2, c3, c4, c5, c6,
          dup0_ref, dup1_ref, dkd_ref, sel_ref, kup0_ref, kup1_ref, kup2_ref,
          embt_ref, emb_ref, phib_ref, bv_ref,
          rec_ref, loss_ref):
    a = pl.program_id(0)
    x = x_ref[0]

    def lin(h, m, b, relu):
        y = _bdot(h, m[0]) + b[0]
        return jnp.maximum(y, 0.0) if relu else y

    h = lin(x, m1, b1, True)
    h = lin(h, m2, b2, True)
    h = lin(h, m3, b3, False)
    h = lin(h, m4, b4, True)
    h = lin(h, m5, b5, False)
    h = lin(h, m6, b6, False)
    f = lin(h, mq, bq, False)

    f_hat = jnp.zeros_like(f)
    f_rest = f
    sse = jnp.float32(0.0)
    kups = [kup0_ref, kup1_ref, kup2_ref]
    pos = 0
    for si, pn in enumerate(_PN):
        last = si == len(_PN) - 1
        es = []
        for hh in range(pn):
            if last:
                z = jnp.dot(f_rest, sel_ref[hh], precision=_HI,
                            preferred_element_type=jnp.float32)
            else:
                z = jnp.dot(f_rest, dkd_ref[pos], precision=_HI,
                            preferred_element_type=jnp.float32)
            nrm = jnp.sqrt(jnp.sum(z * z, axis=1, keepdims=True))
            zn = z / (nrm + 1e-6)
            s = _bdot(zn, embt_ref[0])
            mx = jnp.max(s, axis=1, keepdims=True)
            io = jax.lax.broadcasted_iota(jnp.int32, s.shape, 1)
            idx = jnp.min(jnp.where(s >= mx, io, _V), axis=1, keepdims=True)
            oh = (io == idx).astype(jnp.float32)
            e = jnp.dot(oh, emb_ref[0], precision=_HI,
                        preferred_element_type=jnp.float32)
            es.append(e)
            pos += 1
        e_cat = es[0] if pn == 1 else jnp.concatenate(es, axis=1)
        if last:
            h_up = e_cat
        else:
            h_up = jnp.dot(e_cat, kups[si][:, :], precision=_HI,
                           preferred_element_type=jnp.float32)
        ph = _bdot(h_up, phib_ref[0, si]) + bv_ref[0, si]
        contrib = (1.0 - _RESI) * h_up + _RESI * ph
        f_hat = f_hat + contrib
        f_rest = f_rest - contrib
        dlt = f_hat - f
        sse = sse + jnp.sum(dlt * dlt)

    lc = sse * ((1.0 + _BETA) / len(_PN) / (_B * _H * _C))

    h = lin(f_hat, mpq, bpq, False)
    h = lin(h, d1, c1, True)
    h = lin(h, d2, c2, True)
    h = _bdot(h, dup0_ref[:, :])
    h = lin(h, d3, c3, False)
    h = lin(h, d4, c4, True)
    h = _bdot(h, dup1_ref[:, :])
    h = lin(h, d5, c5, False)
    rec = lin(h, d6, c6, False)

    rec_ref[0] = rec

    lc2 = jnp.reshape(lc, (1, 1))

    @pl.when(a == 0)
    def _init():
        loss_ref[:, :] = lc2

    @pl.when(a != 0)
    def _acc():
        loss_ref[:, :] = loss_ref[:, :] + lc2


def kernel(inp, params):
    enc, dec = params['enc'], params['dec']

    def sw(lst):
        return jnp.stack([p['w'] for p in lst]), jnp.stack([p['b'] for p in lst])

    def bf(m):
        return m.astype(jnp.bfloat16)

    w, b = sw([e['conv_in'] for e in enc])
    m1, b1 = bf(_band(_hmap_s1(16), w)), _btile(b, 16)
    w, b = sw([e['blocks'][0] for e in enc])
    m2, b2 = bf(_band(_hmap_s1(16), w)), _btile(b, 16)
    w, b = sw([e['down'][0] for e in enc])
    m3, b3 = bf(_band(_hmap_s2(16), w)), _btile(b, 8)
    w, b = sw([e['blocks'][1] for e in enc])
    m4, b4 = bf(_band(_hmap_s1(8), w)), _btile(b, 8)
    w, b = sw([e['down'][1] for e in enc])
    m5, b5 = bf(_band(_hmap_s2(8), w)), _btile(b, 4)
    w, b = sw([e['conv_out'] for e in enc])
    m6, b6 = bf(_band(_hmap_s1(4), w)), _btile(b, 4)
    w, b = sw(params['qc'])
    mq, bq = bf(_band(_hmap_s1(4), w)), _btile(b, 4)
    w, b = sw(params['pqc'])
    mpq, bpq = bf(_band(_hmap_s1(4), w)), _btile(b, 4)

    w, b = sw([d['conv_in'] for d in dec])
    d1, c1 = bf(_band(_hmap_s1(4), w)), _btile(b, 4)
    w, b = sw([d['blocks'][0] for d in dec])
    d2, c2 = bf(_band(_hmap_s1(4), w)), _btile(b, 4)
    w, b = sw([d['up'][0] for d in dec])
    d3, c3 = bf(_band(_hmap_s1(8), w)), _btile(b, 8)
    w, b = sw([d['blocks'][1] for d in dec])
    d4, c4 = bf(_band(_hmap_s1(8), w)), _btile(b, 8)
    w, b = sw([d['up'][1] for d in dec])
    d5, c5 = bf(_band(_hmap_s1(16), w)), _btile(b, 16)
    w, b = sw([d['conv_out'] for d in dec])
    d6, c6 = bf(_band(_hmap_s1(16), w)), _btile(b, 16)

    dup0 = bf(jnp.asarray(_dup(4, _C)))
    dup1 = bf(jnp.asarray(_dup(8, _CH)))

    emb = jnp.stack(params['emb'])
    emb_n = emb / (jnp.linalg.norm(emb, axis=2, keepdims=True) + 1e-6)
    embt = bf(jnp.transpose(emb_n, (0, 2, 1)))

    eye_c = jnp.eye(_C, dtype=jnp.float32)
    dkd_list, phib_list, bv_list = [], [], []
    kup_list = []
    for si, pn in enumerate(_PN[:-1]):
        r_dn = jax.image.resize(jnp.eye(_H, dtype=jnp.float32), (pn, _H),
                                method='linear')
        r_up = jax.image.resize(jnp.eye(pn, dtype=jnp.float32), (_H, pn),
                                method='linear')
        for hh in range(pn):
            dkd_list.append(jnp.einsum('i,cd->icd', r_dn[hh],
                                       eye_c).reshape(_H * _C, _C))
        kup_list.append(jnp.einsum('ip,cd->pcid', r_up,
                                   eye_c).reshape(pn * _C, _H * _C))
    sel = jnp.stack([jnp.einsum('i,cd->icd', jnp.eye(_H, dtype=jnp.float32)[hh],
                                eye_c).reshape(_H * _C, _C)
                     for hh in range(_H)])
    dkd = jnp.stack(dkd_list)
    for si in range(len(_PN)):
        w, b = sw([params['phi'][ai][si] for ai in range(_A)])
        phib_list.append(_band(_hmap_s1(_H), w))
        bv_list.append(_btile(b, _H)[:, 0, :])
    phib = bf(jnp.stack(phib_list, axis=1))
    bv = jnp.stack(bv_list, axis=1)

    x = jnp.transpose(inp[:, 0], (2, 0, 1))

    mats = [m1, m2, m3, m4, m5, m6, mq, mpq, d1, d2, d3, d4, d5, d6]
    biases = [b1, b2, b3, b4, b5, b6, bq, bpq, c1, c2, c3, c4, c5, c6]

    def mat_spec(m):
        nd = m.ndim
        return pl.BlockSpec((1,) + m.shape[1:],
                            lambda a, _n=nd: (a,) + (0,) * (_n - 1))

    def full_spec(m):
        nd = m.ndim
        return pl.BlockSpec(m.shape, lambda a, _n=nd: (0,) * _n)

    in_specs = ([pl.BlockSpec((1, _B, _NA), lambda a: (a, 0, 0))]
                + [mat_spec(m) for m in mats]
                + [mat_spec(bb) for bb in biases]
                + [full_spec(dup0), full_spec(dup1), full_spec(dkd),
                   full_spec(sel)]
                + [full_spec(k) for k in kup_list]
                + [mat_spec(embt), mat_spec(emb), mat_spec(phib),
                   mat_spec(bv)])

    rec_all, loss = _pallas_call(
        _body,
        grid=(_A,),
        in_specs=in_specs,
        out_specs=[pl.BlockSpec((1, _B, _NA), lambda a: (a, 0, 0)),
                   pl.BlockSpec((1, 1), lambda a: (0, 0))],
        out_shape=[jax.ShapeDtypeStruct((_A, _B, _NA), jnp.float32),
                   jax.ShapeDtypeStruct((1, 1), jnp.float32)],
    )(x, *mats, *biases, dup0, dup1, dkd, sel, *kup_list,
      embt, emb, phib, bv)

    out = jnp.transpose(rec_all, (1, 2, 0))[:, None]
    return out, loss[0, 0]


# speedup vs baseline: 2.1885x; 1.6126x over previous
import numpy as np
import jax
import jax.numpy as jnp
from jax.experimental import pallas as pl

_B = 1024
_V = 1024
_C = 32
_CH = 16
_A = 10
_NA = 16
_PN = (1, 2, 3, 4)
_BETA = 0.25
_RESI = 0.5
_H = 4

_pallas_call = pl.pallas_call
_HI = jax.lax.Precision.HIGHEST


def _hmap_s1(h):
    s = np.zeros((3, h, h), np.float32)
    for k in range(3):
        for ho in range(h):
            hi = ho + k - 1
            if 0 <= hi < h:
                s[k, hi, ho] = 1.0
    return s


def _hmap_s2(h):
    s = np.zeros((3, h, h // 2), np.float32)
    for k in range(3):
        for ho in range(h // 2):
            hi = 2 * ho + k
            if 0 <= hi < h:
                s[k, hi, ho] = 1.0
    return s


def _dup(h, c):
    d = np.zeros((h * c, 2 * h * c), np.float32)
    for ho in range(2 * h):
        for cc in range(c):
            d[(ho // 2) * c + cc, ho * c + cc] = 1.0
    return d


def _band(smap, w_stack):
    wt = w_stack[:, :, :, :, 1]
    m = jnp.einsum('kio,adck->aicod', jnp.asarray(smap), wt)
    a, hi, ci, ho, co = m.shape
    return m.reshape(a, hi * ci, ho * co)


def _btile(b_stack, ho):
    a, co = b_stack.shape
    return jnp.broadcast_to(b_stack[:, None, :], (a, ho, co)).reshape(a, 1, ho * co)


def _bdot(x, m):
    return jnp.dot(x.astype(jnp.bfloat16), m, preferred_element_type=jnp.float32)


def _body(x_ref,
          m1, m2, m3, m4, m5, m6, mq, mpq, d1, d2, d3, d4, d5, d6,
          b1, b2, b3, b4, b5, b6, bq, bpq, c1, c2, c3, c4, c5, c6,
          dup0_ref, dup1_ref, dkd_ref, kup0_ref, kup1_ref, kup2_ref,
          embt_ref, embp_ref, phib_ref, bv_ref,
          rec_ref, loss_ref):
    a = pl.program_id(0)
    x = x_ref[0]

    def lin(h, m, b, relu):
        y = _bdot(h, m[0]) + b[0]
        return jnp.maximum(y, 0.0) if relu else y

    h = lin(x, m1, b1, True)
    h = lin(h, m2, b2, True)
    h = lin(h, m3, b3, False)
    h = lin(h, m4, b4, True)
    h = lin(h, m5, b5, False)
    h = lin(h, m6, b6, False)
    f = lin(h, mq, bq, False)

    f_hat = jnp.zeros_like(f)
    f_rest = f
    sse = jnp.float32(0.0)
    kups = [kup0_ref, kup1_ref, kup2_ref]
    pos = 0
    for si, pn in enumerate(_PN):
        last = si == len(_PN) - 1
        es = []
        for hh in range(pn):
            if last:
                z = f_rest[:, hh * _C:(hh + 1) * _C]
            else:
                z = jnp.dot(f_rest, dkd_ref[pos], precision=_HI,
                            preferred_element_type=jnp.float32)
            nrm = jnp.sqrt(jnp.sum(z * z, axis=1, keepdims=True))
            zn = z / (nrm + 1e-6)
            s = _bdot(zn, embt_ref[0])
            mx = jnp.max(s, axis=1, keepdims=True)
            io = jax.lax.broadcasted_iota(jnp.int32, s.shape, 1)
            idx = jnp.min(jnp.where(s >= mx, io, _V), axis=1, keepdims=True)
            oh = (io == idx).astype(jnp.bfloat16)
            ep = jnp.dot(oh, embp_ref[0], preferred_element_type=jnp.float32)
            e = (ep[:, :_C] + ep[:, _C:2 * _C]) + ep[:, 2 * _C:]
            es.append(e)
            pos += 1
        e_cat = es[0] if pn == 1 else jnp.concatenate(es, axis=1)
        if last:
            h_up = e_cat
        else:
            h_up = jnp.dot(e_cat, kups[si][:, :], precision=_HI,
                           preferred_element_type=jnp.float32)
        ph = _bdot(h_up, phib_ref[0, si]) + bv_ref[0, si]
        contrib = (1.0 - _RESI) * h_up + _RESI * ph
        f_hat = f_hat + contrib
        f_rest = f_rest - contrib
        dlt = f_hat - f
        sse = sse + jnp.sum(dlt * dlt)

    lc = sse * ((1.0 + _BETA) / len(_PN) / (_B * _H * _C))

    h = lin(f_hat, mpq, bpq, False)
    h = lin(h, d1, c1, True)
    h = lin(h, d2, c2, True)
    h = _bdot(h, dup0_ref[:, :])
    h = lin(h, d3, c3, False)
    h = lin(h, d4, c4, True)
    h = _bdot(h, dup1_ref[:, :])
    h = lin(h, d5, c5, False)
    rec = lin(h, d6, c6, False)

    rec_ref[0] = rec

    lc2 = jnp.reshape(lc, (1, 1))

    @pl.when(a == 0)
    def _init():
        loss_ref[:, :] = lc2

    @pl.when(a != 0)
    def _acc():
        loss_ref[:, :] = loss_ref[:, :] + lc2


def kernel(inp, params):
    enc, dec = params['enc'], params['dec']

    def sw(lst):
        return jnp.stack([p['w'] for p in lst]), jnp.stack([p['b'] for p in lst])

    def bf(m):
        return m.astype(jnp.bfloat16)

    w, b = sw([e['conv_in'] for e in enc])
    m1, b1 = bf(_band(_hmap_s1(16), w)), _btile(b, 16)
    w, b = sw([e['blocks'][0] for e in enc])
    m2, b2 = bf(_band(_hmap_s1(16), w)), _btile(b, 16)
    w, b = sw([e['down'][0] for e in enc])
    m3, b3 = bf(_band(_hmap_s2(16), w)), _btile(b, 8)
    w, b = sw([e['blocks'][1] for e in enc])
    m4, b4 = bf(_band(_hmap_s1(8), w)), _btile(b, 8)
    w, b = sw([e['down'][1] for e in enc])
    m5, b5 = bf(_band(_hmap_s2(8), w)), _btile(b, 4)
    w, b = sw([e['conv_out'] for e in enc])
    m6, b6 = bf(_band(_hmap_s1(4), w)), _btile(b, 4)
    w, b = sw(params['qc'])
    mq, bq = bf(_band(_hmap_s1(4), w)), _btile(b, 4)
    w, b = sw(params['pqc'])
    mpq, bpq = bf(_band(_hmap_s1(4), w)), _btile(b, 4)

    w, b = sw([d['conv_in'] for d in dec])
    d1, c1 = bf(_band(_hmap_s1(4), w)), _btile(b, 4)
    w, b = sw([d['blocks'][0] for d in dec])
    d2, c2 = bf(_band(_hmap_s1(4), w)), _btile(b, 4)
    w, b = sw([d['up'][0] for d in dec])
    d3, c3 = bf(_band(_hmap_s1(8), w)), _btile(b, 8)
    w, b = sw([d['blocks'][1] for d in dec])
    d4, c4 = bf(_band(_hmap_s1(8), w)), _btile(b, 8)
    w, b = sw([d['up'][1] for d in dec])
    d5, c5 = bf(_band(_hmap_s1(16), w)), _btile(b, 16)
    w, b = sw([d['conv_out'] for d in dec])
    d6, c6 = bf(_band(_hmap_s1(16), w)), _btile(b, 16)

    dup0 = bf(jnp.asarray(_dup(4, _C)))
    dup1 = bf(jnp.asarray(_dup(8, _CH)))

    emb = jnp.stack(params['emb'])
    emb_n = emb / (jnp.linalg.norm(emb, axis=2, keepdims=True) + 1e-6)
    embt = bf(jnp.transpose(emb_n, (0, 2, 1)))

    eye_c = jnp.eye(_C, dtype=jnp.float32)
    dkd_list, phib_list, bv_list = [], [], []
    kup_list = []
    for si, pn in enumerate(_PN[:-1]):
        r_dn = jax.image.resize(jnp.eye(_H, dtype=jnp.float32), (pn, _H),
                                method='linear')
        r_up = jax.image.resize(jnp.eye(pn, dtype=jnp.float32), (_H, pn),
                                method='linear')
        for hh in range(pn):
            dkd_list.append(jnp.einsum('i,cd->icd', r_dn[hh],
                                       eye_c).reshape(_H * _C, _C))
        kup_list.append(jnp.einsum('ip,cd->pcid', r_up,
                                   eye_c).reshape(pn * _C, _H * _C))
    dkd = jnp.stack(dkd_list)
    e0 = emb.astype(jnp.bfloat16)
    r1 = emb - e0.astype(jnp.float32)
    e1 = r1.astype(jnp.bfloat16)
    e2 = (r1 - e1.astype(jnp.float32)).astype(jnp.bfloat16)
    embp = jnp.concatenate([e0, e1, e2], axis=2)
    for si in range(len(_PN)):
        w, b = sw([params['phi'][ai][si] for ai in range(_A)])
        phib_list.append(_band(_hmap_s1(_H), w))
        bv_list.append(_btile(b, _H)[:, 0, :])
    phib = bf(jnp.stack(phib_list, axis=1))
    bv = jnp.stack(bv_list, axis=1)

    x = jnp.transpose(inp[:, 0], (2, 0, 1))

    mats = [m1, m2, m3, m4, m5, m6, mq, mpq, d1, d2, d3, d4, d5, d6]
    biases = [b1, b2, b3, b4, b5, b6, bq, bpq, c1, c2, c3, c4, c5, c6]

    def mat_spec(m):
        nd = m.ndim
        return pl.BlockSpec((1,) + m.shape[1:],
                            lambda a, _n=nd: (a,) + (0,) * (_n - 1))

    def full_spec(m):
        nd = m.ndim
        return pl.BlockSpec(m.shape, lambda a, _n=nd: (0,) * _n)

    in_specs = ([pl.BlockSpec((1, _B, _NA), lambda a: (a, 0, 0))]
                + [mat_spec(m) for m in mats]
                + [mat_spec(bb) for bb in biases]
                + [full_spec(dup0), full_spec(dup1), full_spec(dkd)]
                + [full_spec(k) for k in kup_list]
                + [mat_spec(embt), mat_spec(embp), mat_spec(phib),
                   mat_spec(bv)])

    rec_all, loss = _pallas_call(
        _body,
        grid=(_A,),
        in_specs=in_specs,
        out_specs=[pl.BlockSpec((1, _B, _NA), lambda a: (a, 0, 0)),
                   pl.BlockSpec((1, 1), lambda a: (0, 0))],
        out_shape=[jax.ShapeDtypeStruct((_A, _B, _NA), jnp.float32),
                   jax.ShapeDtypeStruct((1, 1), jnp.float32)],
    )(x, *mats, *biases, dup0, dup1, dkd, *kup_list,
      embt, embp, phib, bv)

    out = jnp.transpose(rec_all, (1, 2, 0))[:, None]
    return out, loss[0, 0]
